# baseline (speedup 1.0000x reference)
import functools

import jax
import jax.numpy as jnp
from jax import lax
from jax.experimental import pallas as pl
from jax.experimental.pallas import tpu as pltpu
from jax.experimental.pallas import tpu_sc as plsc

VOCAB = 1000
VOCAB_PAD = 1008
BATCH = 16384
NUM_CORES = 2
NUM_SUBCORES = 16
LANES = 16
NUM_WORKERS = NUM_CORES * NUM_SUBCORES
B_PER_W = BATCH // NUM_WORKERS

_mesh = plsc.VectorSubcoreMesh(core_axis_name="c", subcore_axis_name="s",
                               num_cores=NUM_CORES)


@functools.partial(
    pl.kernel,
    mesh=_mesh,
    out_type=jax.ShapeDtypeStruct((BATCH,), jnp.float32),
    scratch_types=[
        pltpu.VMEM((VOCAB,), jnp.float32),
        pltpu.VMEM((B_PER_W,), jnp.int32),
        pltpu.VMEM((B_PER_W,), jnp.float32),
        pltpu.SemaphoreType.DMA,
        pltpu.SemaphoreType.DMA,
    ],
    compiler_params=pltpu.CompilerParams(needs_layout_passes=False),
)
def _ig_gather_sigmoid(genes_hbm, ig_hbm, out_hbm, tab_v, idx_v, out_v,
                       sem_in, sem_out):
    wid = lax.axis_index("s") * NUM_CORES + lax.axis_index("c")
    base = wid * B_PER_W
    tab_cp = pltpu.async_copy(ig_hbm, tab_v, sem_in)
    idx_cp = pltpu.async_copy(genes_hbm.at[pl.ds(base, B_PER_W)], idx_v,
                              sem_in)
    tab_cp.wait()
    idx_cp.wait()

    def step(off):
        idx = idx_v[pl.ds(off, LANES)]
        g = plsc.load_gather(tab_v, [idx])
        out_v[pl.ds(off, LANES)] = 1.0 / (1.0 + jnp.exp(-g))

    plsc.parallel_loop(0, B_PER_W, step=LANES, unroll=3)(step)
    out_cp = pltpu.async_copy(out_v, out_hbm.at[pl.ds(base, B_PER_W)],
                              sem_out)
    out_cp.wait()


def kernel(current_genes, ig):
    return _ig_gather_sigmoid(current_genes.astype(jnp.int32), ig)


# speedup vs baseline: 1.0940x; 1.0940x over previous
import functools

import jax
import jax.numpy as jnp
from jax import lax
from jax.experimental import pallas as pl
from jax.experimental.pallas import tpu as pltpu
from jax.experimental.pallas import tpu_sc as plsc

VOCAB = 1000
VOCAB_PAD = 1008
BATCH = 16384
NUM_CORES = 1
NUM_SUBCORES = 16
LANES = 16
NUM_WORKERS = NUM_CORES * NUM_SUBCORES
B_PER_W = BATCH // NUM_WORKERS

_mesh = plsc.VectorSubcoreMesh(core_axis_name="c", subcore_axis_name="s",
                               num_cores=NUM_CORES)


@functools.partial(
    pl.kernel,
    mesh=_mesh,
    out_type=jax.ShapeDtypeStruct((BATCH,), jnp.float32),
    scratch_types=[
        pltpu.VMEM((VOCAB,), jnp.float32),
        pltpu.VMEM((B_PER_W,), jnp.int32),
        pltpu.VMEM((B_PER_W,), jnp.float32),
        pltpu.SemaphoreType.DMA,
        pltpu.SemaphoreType.DMA,
    ],
    compiler_params=pltpu.CompilerParams(needs_layout_passes=False),
)
def _ig_gather_sigmoid(genes_hbm, ig_hbm, out_hbm, tab_v, idx_v, out_v,
                       sem_in, sem_out):
    wid = lax.axis_index("s") * NUM_CORES + lax.axis_index("c")
    base = wid * B_PER_W
    tab_cp = pltpu.async_copy(ig_hbm, tab_v, sem_in)
    idx_cp = pltpu.async_copy(genes_hbm.at[pl.ds(base, B_PER_W)], idx_v,
                              sem_in)
    tab_cp.wait()
    idx_cp.wait()

    def step(off):
        idx = idx_v[pl.ds(off, LANES)]
        g = plsc.load_gather(tab_v, [idx])
        out_v[pl.ds(off, LANES)] = 1.0 / (1.0 + jnp.exp(-g))

    plsc.parallel_loop(0, B_PER_W, step=LANES, unroll=4)(step)
    out_cp = pltpu.async_copy(out_v, out_hbm.at[pl.ds(base, B_PER_W)],
                              sem_out)
    out_cp.wait()


def kernel(current_genes, ig):
    return _ig_gather_sigmoid(current_genes.astype(jnp.int32), ig)
